# baseline (speedup 1.0000x reference)
import functools

import numpy as np
import jax
import jax.numpy as jnp
from jax import lax
from jax.experimental import pallas as pl
from jax.experimental.pallas import tpu as pltpu
from jax.experimental.pallas import tpu_sc as plsc

_VOCAB = 100000
_DIM = 128
_SEQ = 200
_BATCH = 4096

_NC = 2
_NS = 16
_NW = _NC * _NS
_ROWS = _BATCH * _SEQ
_RPW = _ROWS // _NW
_C = 80
_NCHUNK = _RPW // _C
_PE2R = 240
_NG = 5
_NO = 2
_NI = 10
_IDXAHEAD = 8


def _pos_encoding() -> np.ndarray:
    pos = np.arange(_SEQ, dtype=np.float32)[:, None]
    div = np.exp(np.arange(0, _DIM, 2, dtype=np.float32) * (-np.log(10000.0) / _DIM))
    pe = np.zeros((_SEQ, _DIM), dtype=np.float32)
    pe[:, 0::2] = np.sin(pos * div)
    pe[:, 1::2] = np.cos(pos * div)
    return pe


_PE = _pos_encoding()
_PE2 = np.concatenate([_PE, _PE[: _PE2R - _SEQ]], axis=0)


def _make_sc_kernel():
    mesh = plsc.VectorSubcoreMesh(core_axis_name="c", subcore_axis_name="s")

    scratch = (
        [pltpu.VMEM((_NI, _C), jnp.int32)]
        + [pltpu.VMEM((_C, _DIM), jnp.float32) for _ in range(_NG + _NO)]
        + [pltpu.VMEM((_PE2R, _DIM), jnp.float32)]
        + [pltpu.SemaphoreType.DMA for _ in range(_NG + _NO + _NI)]
    )

    @functools.partial(
        pl.kernel,
        mesh=mesh,
        out_type=jax.ShapeDtypeStruct((_ROWS, _DIM), jnp.float32),
        scratch_types=scratch,
    )
    def body(xf, table, pe, out, idx_v, *rest):
        rows_b = rest[:_NG]
        st_b = rest[_NG:_NG + _NO]
        pe_v = rest[_NG + _NO]
        sems = rest[_NG + _NO + 1:]
        gsem = sems[:_NG]
        osem = sems[_NG:_NG + _NO]
        isem = sems[_NG + _NO:]

        wid = lax.axis_index("s") * _NC + lax.axis_index("c")
        base = wid * _RPW

        pltpu.sync_copy(pe, pe_v)

        def fire_idx(g, ib):
            pltpu.async_copy(xf.at[wid, g], idx_v.at[ib], isem[ib])

        def fire(g, gb, ib):
            pltpu.make_async_copy(xf.at[wid, g], idx_v.at[ib], isem[ib]).wait()
            pltpu.async_copy(table.at[idx_v.at[ib]], rows_b[gb], gsem[gb])

        for _g in range(_IDXAHEAD):
            fire_idx(_g, _g)
        for _g in range(_NG - 1):
            fire(_g, _g, _g)

        def compute(g, gb, ob):
            src = rows_b[gb]
            dst = st_b[ob]
            s_start = lax.rem(g * _C, _SEQ)

            @plsc.parallel_loop(0, _C, unroll=16)
            def row(r):
                s = s_start + r
                for d in range(_DIM // 16):
                    sl = pl.ds(d * 16, 16)
                    e = src[r, sl]
                    dst[r, sl] = e + e + pe_v[s, sl]

        def step(g, gb, ob, ib):
            pltpu.make_async_copy(table.at[idx_v.at[ib]], rows_b[gb], gsem[gb]).wait()

            @pl.when(g + _NG - 1 < _NCHUNK)
            def _fire_next():
                fire(g + _NG - 1, (gb + _NG - 1) % _NG, (ib + _NG - 1) % _NI)

            @pl.when(g + _IDXAHEAD < _NCHUNK)
            def _fire_next_idx():
                fire_idx(g + _IDXAHEAD, (ib + _IDXAHEAD) % _NI)

            @pl.when(g >= _NO)
            def _wait_prev_out():
                pltpu.make_async_copy(
                    st_b[ob], out.at[pl.ds(base + (g - _NO) * _C, _C)], osem[ob]
                ).wait()

            compute(g, gb, ob)
            pltpu.async_copy(st_b[ob], out.at[pl.ds(base + g * _C, _C)], osem[ob])

        def group(i, carry):
            for b in range(_NI):
                step(_NI * i + b, b % _NG, b % _NO, b % _NI)
            return carry

        lax.fori_loop(0, _NCHUNK // _NI, group, None)

        for k in range(_NO):
            g = _NCHUNK - _NO + k
            pltpu.make_async_copy(
                st_b[g % _NO], out.at[pl.ds(base + g * _C, _C)], osem[g % _NO]
            ).wait()

    return body


def kernel(x, table):
    xf = x.reshape(_NW, _NCHUNK, _C)
    pe = jnp.asarray(_PE2)
    out = _make_sc_kernel()(xf, table, pe)
    return out.reshape(_BATCH, _SEQ, _DIM)


# speedup vs baseline: 1.3638x; 1.3638x over previous
import functools

import numpy as np
import jax
import jax.numpy as jnp
from jax import lax
from jax.experimental import pallas as pl
from jax.experimental.pallas import tpu as pltpu
from jax.experimental.pallas import tpu_sc as plsc

_VOCAB = 100000
_DIM = 128
_SEQ = 200
_BATCH = 4096

_NC = 2
_NS = 16
_NW = _NC * _NS
_ROWS = _BATCH * _SEQ
_RPW = _ROWS // _NW
_C = 80
_NCHUNK = _RPW // _C
_PE2R = 240
_NG = 5
_NO = 2
_NI = 10
_IDXAHEAD = 8


def _pos_encoding() -> np.ndarray:
    pos = np.arange(_SEQ, dtype=np.float32)[:, None]
    div = np.exp(np.arange(0, _DIM, 2, dtype=np.float32) * (-np.log(10000.0) / _DIM))
    pe = np.zeros((_SEQ, _DIM), dtype=np.float32)
    pe[:, 0::2] = np.sin(pos * div)
    pe[:, 1::2] = np.cos(pos * div)
    return pe


_PE = _pos_encoding()
_PE2 = np.concatenate([_PE, _PE[: _PE2R - _SEQ]], axis=0)


def _make_sc_kernel():
    mesh = plsc.VectorSubcoreMesh(core_axis_name="c", subcore_axis_name="s")

    scratch = (
        [pltpu.VMEM((_NI, _C), jnp.int32)]
        + [pltpu.VMEM((_C, _DIM), jnp.float32) for _ in range(_NG + _NO)]
        + [pltpu.VMEM((_PE2R, _DIM), jnp.float32)]
        + [pltpu.SemaphoreType.DMA for _ in range(_NG + _NO + _NI)]
    )

    @functools.partial(
        pl.kernel,
        mesh=mesh,
        out_type=jax.ShapeDtypeStruct((_ROWS, _DIM), jnp.float32),
        scratch_types=scratch,
    )
    def body(xf, table, pe, out, idx_v, *rest):
        rows_b = rest[:_NG]
        st_b = rest[_NG:_NG + _NO]
        pe_v = rest[_NG + _NO]
        sems = rest[_NG + _NO + 1:]
        gsem = sems[:_NG]
        osem = sems[_NG:_NG + _NO]
        isem = sems[_NG + _NO:]

        wid = lax.axis_index("s") * _NC + lax.axis_index("c")
        base = wid * _RPW

        pltpu.sync_copy(pe, pe_v)

        def fire_idx(g, ib):
            pltpu.async_copy(xf.at[wid, g], idx_v.at[ib], isem[ib])

        def fire(g, gb, ib):
            pltpu.make_async_copy(xf.at[wid, g], idx_v.at[ib], isem[ib]).wait()
            pltpu.async_copy(table.at[idx_v.at[ib]], rows_b[gb], gsem[gb])

        for _g in range(_IDXAHEAD):
            fire_idx(_g, _g)
        for _g in range(_NG - 1):
            fire(_g, _g, _g)

        def compute(g, gb, ob):
            src = rows_b[gb]
            dst = st_b[ob]
            s_start = lax.rem(g * _C, _SEQ)

            @plsc.parallel_loop(0, _C, unroll=10)
            def row(r):
                s = s_start + r
                for d in range(_DIM // 16):
                    sl = pl.ds(d * 16, 16)
                    e = src[r, sl]
                    dst[r, sl] = e + e + pe_v[s, sl]

        def step(g, gb, ob, ib):
            pltpu.make_async_copy(table.at[idx_v.at[ib]], rows_b[gb], gsem[gb]).wait()

            @pl.when(g + _NG - 1 < _NCHUNK)
            def _fire_next():
                fire(g + _NG - 1, (gb + _NG - 1) % _NG, (ib + _NG - 1) % _NI)

            @pl.when(g + _IDXAHEAD < _NCHUNK)
            def _fire_next_idx():
                fire_idx(g + _IDXAHEAD, (ib + _IDXAHEAD) % _NI)

            @pl.when(g >= _NO)
            def _wait_prev_out():
                pltpu.make_async_copy(
                    st_b[ob], out.at[pl.ds(base + (g - _NO) * _C, _C)], osem[ob]
                ).wait()

            compute(g, gb, ob)
            pltpu.async_copy(st_b[ob], out.at[pl.ds(base + g * _C, _C)], osem[ob])

        def group(i, carry):
            for b in range(_NI):
                step(_NI * i + b, b % _NG, b % _NO, b % _NI)
            return carry

        lax.fori_loop(0, _NCHUNK // _NI, group, None)

        for k in range(_NO):
            g = _NCHUNK - _NO + k
            pltpu.make_async_copy(
                st_b[g % _NO], out.at[pl.ds(base + g * _C, _C)], osem[g % _NO]
            ).wait()

    return body


def kernel(x, table):
    xf = x.reshape(_NW, _NCHUNK, _C)
    pe = jnp.asarray(_PE2)
    out = _make_sc_kernel()(xf, table, pe)
    return out.reshape(_BATCH, _SEQ, _DIM)


# speedup vs baseline: 1.4527x; 1.0652x over previous
import functools

import numpy as np
import jax
import jax.numpy as jnp
from jax import lax
from jax.experimental import pallas as pl
from jax.experimental.pallas import tpu as pltpu
from jax.experimental.pallas import tpu_sc as plsc

_VOCAB = 100000
_DIM = 128
_SEQ = 200
_BATCH = 4096

_NC = 2
_NS = 16
_NW = _NC * _NS
_ROWS = _BATCH * _SEQ
_RPW = _ROWS // _NW
_C = 80
_NCHUNK = _RPW // _C
_PE2R = 240
_NG = 5
_NO = 2
_NI = 10
_IDXAHEAD = 8


def _pos_encoding() -> np.ndarray:
    pos = np.arange(_SEQ, dtype=np.float32)[:, None]
    div = np.exp(np.arange(0, _DIM, 2, dtype=np.float32) * (-np.log(10000.0) / _DIM))
    pe = np.zeros((_SEQ, _DIM), dtype=np.float32)
    pe[:, 0::2] = np.sin(pos * div)
    pe[:, 1::2] = np.cos(pos * div)
    return pe


_PE = _pos_encoding()
_PE2 = np.concatenate([_PE, _PE[: _PE2R - _SEQ]], axis=0)


def _make_sc_kernel():
    mesh = plsc.VectorSubcoreMesh(core_axis_name="c", subcore_axis_name="s")

    scratch = (
        [pltpu.VMEM((_NI, _C), jnp.int32)]
        + [pltpu.VMEM((_C, _DIM), jnp.float32) for _ in range(_NG + _NO)]
        + [pltpu.VMEM((_PE2R, _DIM), jnp.float32)]
        + [pltpu.SemaphoreType.DMA for _ in range(_NG + _NO + _NI)]
    )

    @functools.partial(
        pl.kernel,
        mesh=mesh,
        out_type=jax.ShapeDtypeStruct((_ROWS, _DIM), jnp.float32),
        scratch_types=scratch,
    )
    def body(xf, table, pe, out, idx_v, *rest):
        rows_b = rest[:_NG]
        st_b = rest[_NG:_NG + _NO]
        pe_v = rest[_NG + _NO]
        sems = rest[_NG + _NO + 1:]
        gsem = sems[:_NG]
        osem = sems[_NG:_NG + _NO]
        isem = sems[_NG + _NO:]

        wid = lax.axis_index("s") * _NC + lax.axis_index("c")
        base = wid * _RPW

        pltpu.sync_copy(pe, pe_v)

        def fire_idx(g, ib):
            pltpu.async_copy(xf.at[wid, g], idx_v.at[ib], isem[ib])

        def fire(g, gb, ib):
            pltpu.make_async_copy(xf.at[wid, g], idx_v.at[ib], isem[ib]).wait()
            pltpu.async_copy(table.at[idx_v.at[ib]], rows_b[gb], gsem[gb])

        for _g in range(_IDXAHEAD):
            fire_idx(_g, _g)
        for _g in range(_NG - 1):
            fire(_g, _g, _g)

        def compute(g, gb, ob):
            src = rows_b[gb]
            dst = st_b[ob]
            s_start = lax.rem(g * _C, _SEQ)

            @plsc.parallel_loop(0, _C, unroll=8)
            def row(r):
                s = s_start + r
                for d in range(_DIM // 16):
                    sl = pl.ds(d * 16, 16)
                    e = src[r, sl]
                    dst[r, sl] = e + e + pe_v[s, sl]

        def step(g, gb, ob, ib):
            pltpu.make_async_copy(table.at[idx_v.at[ib]], rows_b[gb], gsem[gb]).wait()

            @pl.when(g + _NG - 1 < _NCHUNK)
            def _fire_next():
                fire(g + _NG - 1, (gb + _NG - 1) % _NG, (ib + _NG - 1) % _NI)

            @pl.when(g + _IDXAHEAD < _NCHUNK)
            def _fire_next_idx():
                fire_idx(g + _IDXAHEAD, (ib + _IDXAHEAD) % _NI)

            @pl.when(g >= _NO)
            def _wait_prev_out():
                pltpu.make_async_copy(
                    st_b[ob], out.at[pl.ds(base + (g - _NO) * _C, _C)], osem[ob]
                ).wait()

            compute(g, gb, ob)
            pltpu.async_copy(st_b[ob], out.at[pl.ds(base + g * _C, _C)], osem[ob])

        def group(i, carry):
            for b in range(_NI):
                step(_NI * i + b, b % _NG, b % _NO, b % _NI)
            return carry

        lax.fori_loop(0, _NCHUNK // _NI, group, None)

        for k in range(_NO):
            g = _NCHUNK - _NO + k
            pltpu.make_async_copy(
                st_b[g % _NO], out.at[pl.ds(base + g * _C, _C)], osem[g % _NO]
            ).wait()

    return body


def kernel(x, table):
    xf = x.reshape(_NW, _NCHUNK, _C)
    pe = jnp.asarray(_PE2)
    out = _make_sc_kernel()(xf, table, pe)
    return out.reshape(_BATCH, _SEQ, _DIM)


# speedup vs baseline: 1.4583x; 1.0039x over previous
import functools

import numpy as np
import jax
import jax.numpy as jnp
from jax import lax
from jax.experimental import pallas as pl
from jax.experimental.pallas import tpu as pltpu
from jax.experimental.pallas import tpu_sc as plsc

_VOCAB = 100000
_DIM = 128
_SEQ = 200
_BATCH = 4096

_NC = 2
_NS = 16
_NW = _NC * _NS
_ROWS = _BATCH * _SEQ
_RPW = _ROWS // _NW
_C = 80
_NCHUNK = _RPW // _C
_PE2R = 240
_NG = 5
_NO = 2
_NI = 10
_IDXAHEAD = 8


def _pos_encoding() -> np.ndarray:
    pos = np.arange(_SEQ, dtype=np.float32)[:, None]
    div = np.exp(np.arange(0, _DIM, 2, dtype=np.float32) * (-np.log(10000.0) / _DIM))
    pe = np.zeros((_SEQ, _DIM), dtype=np.float32)
    pe[:, 0::2] = np.sin(pos * div)
    pe[:, 1::2] = np.cos(pos * div)
    return pe


_PE = _pos_encoding()
_PE2 = np.concatenate([_PE, _PE[: _PE2R - _SEQ]], axis=0)


def _make_sc_kernel():
    mesh = plsc.VectorSubcoreMesh(core_axis_name="c", subcore_axis_name="s")

    scratch = (
        [pltpu.VMEM((_NI, _C), jnp.int32)]
        + [pltpu.VMEM((_C, _DIM), jnp.float32) for _ in range(_NG + _NO)]
        + [pltpu.VMEM((_PE2R, _DIM), jnp.float32)]
        + [pltpu.SemaphoreType.DMA for _ in range(_NG + _NO + _NI)]
    )

    @functools.partial(
        pl.kernel,
        mesh=mesh,
        out_type=jax.ShapeDtypeStruct((_ROWS, _DIM), jnp.float32),
        scratch_types=scratch,
    )
    def body(xf, table, pe, out, idx_v, *rest):
        rows_b = rest[:_NG]
        st_b = rest[_NG:_NG + _NO]
        pe_v = rest[_NG + _NO]
        sems = rest[_NG + _NO + 1:]
        gsem = sems[:_NG]
        osem = sems[_NG:_NG + _NO]
        isem = sems[_NG + _NO:]

        wid = lax.axis_index("s") * _NC + lax.axis_index("c")
        base = wid * _RPW

        pltpu.sync_copy(pe, pe_v)

        def fire_idx(g, ib):
            pltpu.async_copy(xf.at[wid, g], idx_v.at[ib], isem[ib])

        def fire(g, gb, ib):
            pltpu.make_async_copy(xf.at[wid, g], idx_v.at[ib], isem[ib]).wait()
            pltpu.async_copy(table.at[idx_v.at[ib]], rows_b[gb], gsem[gb])

        for _g in range(_IDXAHEAD):
            fire_idx(_g, _g)
        for _g in range(_NG - 1):
            fire(_g, _g, _g)

        def compute(g, gb, ob):
            src = rows_b[gb]
            dst = st_b[ob]
            s_start = lax.rem(g * _C, _SEQ)

            def one_row(r):
                s = s_start + r
                for d in range(_DIM // 16):
                    sl = pl.ds(d * 16, 16)
                    e = src[r, sl]
                    dst[r, sl] = e + e + pe_v[s, sl]

            one_row(0)

            @plsc.parallel_loop(1, _C - 1, unroll=6)
            def row(r):
                one_row(r)

            one_row(_C - 1)

        def step(g, gb, ob, ib):
            pltpu.make_async_copy(table.at[idx_v.at[ib]], rows_b[gb], gsem[gb]).wait()

            @pl.when(g + _NG - 1 < _NCHUNK)
            def _fire_next():
                fire(g + _NG - 1, (gb + _NG - 1) % _NG, (ib + _NG - 1) % _NI)

            @pl.when(g + _IDXAHEAD < _NCHUNK)
            def _fire_next_idx():
                fire_idx(g + _IDXAHEAD, (ib + _IDXAHEAD) % _NI)

            @pl.when(g >= _NO)
            def _wait_prev_out():
                pltpu.make_async_copy(
                    st_b[ob], out.at[pl.ds(base + (g - _NO) * _C, _C)], osem[ob]
                ).wait()

            compute(g, gb, ob)
            pltpu.async_copy(st_b[ob], out.at[pl.ds(base + g * _C, _C)], osem[ob])

        def group(i, carry):
            for b in range(_NI):
                step(_NI * i + b, b % _NG, b % _NO, b % _NI)
            return carry

        lax.fori_loop(0, _NCHUNK // _NI, group, None)

        for k in range(_NO):
            g = _NCHUNK - _NO + k
            pltpu.make_async_copy(
                st_b[g % _NO], out.at[pl.ds(base + g * _C, _C)], osem[g % _NO]
            ).wait()

    return body


def kernel(x, table):
    xf = x.reshape(_NW, _NCHUNK, _C)
    pe = jnp.asarray(_PE2)
    out = _make_sc_kernel()(xf, table, pe)
    return out.reshape(_BATCH, _SEQ, _DIM)
